# baseline (speedup 1.0000x reference)
import functools

import jax
import jax.numpy as jnp
from jax import lax
from jax.experimental import pallas as pl
from jax.experimental.pallas import tpu as pltpu
from jax.experimental.pallas import tpu_sc as plsc

_N_PIX = 512 * 512
_N_SUB = _N_PIX // 2
_N_WORKERS = 32
_PIX_PER = _N_PIX // _N_WORKERS
_N_PAD = 8192
_ICHUNK = 4096
_N_ICHUNKS = _N_SUB // _ICHUNK
_GCHUNK = 1024
_N_GCHUNKS = _PIX_PER // _GCHUNK
_GSUB = 128
_TBLK = 512
_N_TBLK = _N_SUB // _TBLK
_N_ZBLK = _N_PAD // _TBLK


def _tc_transpose_pad(img):
    def body(i_ref, o_ref):
        i = pl.program_id(0)

        @pl.when(i < _N_TBLK)
        def _():
            o_ref[...] = i_ref[...].T

        @pl.when(i >= _N_TBLK)
        def _():
            o_ref[...] = jnp.zeros((_TBLK, 16), jnp.float32)

    return pl.pallas_call(
        body,
        grid=(_N_TBLK + _N_ZBLK,),
        in_specs=[pl.BlockSpec((16, _TBLK),
                               lambda i: (0, jnp.minimum(i, _N_TBLK - 1)))],
        out_specs=pl.BlockSpec((_TBLK, 16), lambda i: (i, 0)),
        out_shape=jax.ShapeDtypeStruct((_N_SUB + _N_PAD, 16), jnp.float32),
    )(img)


def _sc_claim(idx):
    mesh = plsc.VectorSubcoreMesh(core_axis_name="c", subcore_axis_name="s")

    @functools.partial(
        pl.kernel,
        mesh=mesh,
        out_type=jax.ShapeDtypeStruct((_N_WORKERS, _PIX_PER), jnp.int32),
        scratch_types=[
            pltpu.VMEM((_PIX_PER,), jnp.int32),
            pltpu.VMEM((_ICHUNK,), jnp.int32),
            pltpu.VMEM((_ICHUNK,), jnp.int32),
            pltpu.SemaphoreType.DMA,
            pltpu.SemaphoreType.DMA,
            pltpu.SemaphoreType.DMA,
        ],
        compiler_params=pltpu.CompilerParams(
            needs_layout_passes=False, use_tc_tiling_on_sc=False),
    )
    def body(idx_hbm, claim_hbm, claim_v, ibuf0, ibuf1, isem0, isem1, osem):
        wid = lax.axis_index("s") * 2 + lax.axis_index("c")
        base = wid * _PIX_PER
        lanes = lax.iota(jnp.int32, 16)

        def init_body(v, _):
            claim_v[pl.ds(v * 16, 16)] = lanes + (_N_SUB + v * 16)
            return _
        lax.fori_loop(0, _PIX_PER // 16, init_body, 0)

        ibufs = [ibuf0, ibuf1]
        isems = [isem0, isem1]
        pltpu.async_copy(idx_hbm.at[pl.ds(0, _ICHUNK)], ibuf0, isem0)
        for c in range(_N_ICHUNKS):
            par = c % 2
            if c + 1 < _N_ICHUNKS:
                nxt = (c + 1) % 2
                pltpu.async_copy(
                    idx_hbm.at[pl.ds((c + 1) * _ICHUNK, _ICHUNK)],
                    ibufs[nxt], isems[nxt])
            pltpu.make_async_copy(
                idx_hbm.at[pl.ds(c * _ICHUNK, _ICHUNK)],
                ibufs[par], isems[par]).wait()
            ibuf = ibufs[par]

            def scan_body(u, _, c=c, ibuf=ibuf):
                for k in range(4):
                    v = u * 4 + k
                    iv = ibuf[pl.ds(v * 16, 16)]
                    m = (iv >= base) & (iv < base + _PIX_PER)
                    local = (iv - base) & (_PIX_PER - 1)
                    jv = lanes + (c * _ICHUNK + v * 16)
                    plsc.store_scatter(claim_v, [local], jv, mask=m)
                return _
            lax.fori_loop(0, _ICHUNK // 64, scan_body, 0)

        pltpu.async_copy(claim_v, claim_hbm.at[wid], osem)
        pltpu.make_async_copy(claim_v, claim_hbm.at[wid], osem).wait()

    return body(idx)


def _sc_gather(claim, imgT):
    mesh = plsc.VectorSubcoreMesh(core_axis_name="c", subcore_axis_name="s")

    @functools.partial(
        pl.kernel,
        mesh=mesh,
        out_type=jax.ShapeDtypeStruct((16, _N_PIX), jnp.float32),
        scratch_types=[
            pltpu.VMEM((_PIX_PER,), jnp.int32),
            pltpu.VMEM((_GCHUNK, 16), jnp.float32),
            pltpu.VMEM((_GCHUNK, 16), jnp.float32),
            pltpu.VMEM((_GCHUNK * 16,), jnp.float32),
            pltpu.VMEM((_GCHUNK * 16,), jnp.float32),
            pltpu.SemaphoreType.DMA,
            pltpu.SemaphoreType.DMA,
            pltpu.SemaphoreType.DMA,
            pltpu.SemaphoreType.DMA,
            pltpu.SemaphoreType.DMA,
        ],
        compiler_params=pltpu.CompilerParams(
            needs_layout_passes=False, use_tc_tiling_on_sc=False),
    )
    def body(claim_hbm, imgT_hbm, out_hbm, claim_v,
             gbuf0, gbuf1, tbuf0, tbuf1,
             csem, gsem0, gsem1, osem0, osem1):
        wid = lax.axis_index("s") * 2 + lax.axis_index("c")
        base = wid * _PIX_PER
        lanes = lax.iota(jnp.int32, 16)
        lanes_sc = lanes * _GCHUNK

        pltpu.async_copy(claim_hbm.at[wid], claim_v, csem)
        pltpu.make_async_copy(claim_hbm.at[wid], claim_v, csem).wait()

        gbufs = [gbuf0, gbuf1]
        gsems = [gsem0, gsem1]
        tbufs = [tbuf0, tbuf1]
        osems = [osem0, osem1]

        def fire_gathers(c2):
            par = c2 % 2
            for g in range(_GCHUNK // _GSUB):
                pltpu.async_copy(
                    imgT_hbm.at[claim_v.at[pl.ds(c2 * _GCHUNK + g * _GSUB,
                                                 _GSUB)]],
                    gbufs[par].at[pl.ds(g * _GSUB, _GSUB)], gsems[par])

        def drain_gathers(c2):
            par = c2 % 2
            for g in range(_GCHUNK // _GSUB):
                pltpu.make_async_copy(
                    imgT_hbm.at[claim_v.at[pl.ds(c2 * _GCHUNK + g * _GSUB,
                                                 _GSUB)]],
                    gbufs[par].at[pl.ds(g * _GSUB, _GSUB)], gsems[par]).wait()

        def fire_out(c2):
            par = c2 % 2
            for b in range(16):
                pltpu.async_copy(
                    tbufs[par].at[pl.ds(b * _GCHUNK, _GCHUNK)],
                    out_hbm.at[b, pl.ds(base + c2 * _GCHUNK, _GCHUNK)],
                    osems[par])

        def drain_out(c2):
            par = c2 % 2
            for b in range(16):
                pltpu.make_async_copy(
                    tbufs[par].at[pl.ds(b * _GCHUNK, _GCHUNK)],
                    out_hbm.at[b, pl.ds(base + c2 * _GCHUNK, _GCHUNK)],
                    osems[par]).wait()

        fire_gathers(0)
        for c2 in range(_N_GCHUNKS):
            if c2 + 1 < _N_GCHUNKS:
                fire_gathers(c2 + 1)
            drain_gathers(c2)
            if c2 >= 2:
                drain_out(c2 - 2)

            gbuf = gbufs[c2 % 2]
            tbuf = tbufs[c2 % 2]

            def tr_body(u, _, gbuf=gbuf, tbuf=tbuf):
                for k in range(4):
                    p = u * 4 + k
                    row = gbuf[p]
                    plsc.store_scatter(tbuf, [lanes_sc + p], row)
                return _
            lax.fori_loop(0, _GCHUNK // 4, tr_body, 0)
            fire_out(c2)
        drain_out(_N_GCHUNKS - 2)
        drain_out(_N_GCHUNKS - 1)

    return body(claim, imgT)


def kernel(img, subsample_idx, height, width):
    idx = (subsample_idx + (height - 512) + (width - 512)).astype(jnp.int32)
    claim = _sc_claim(idx)
    imgT = _tc_transpose_pad(img)
    out = _sc_gather(claim, imgT)
    return out.reshape(img.shape[0], 1, 512, 512)


# speedup vs baseline: 1.1708x; 1.1708x over previous
import functools

import jax
import jax.numpy as jnp
from jax import lax
from jax.experimental import pallas as pl
from jax.experimental.pallas import tpu as pltpu
from jax.experimental.pallas import tpu_sc as plsc

_N_PIX = 512 * 512
_N_SUB = _N_PIX // 2
_N_WORKERS = 32
_PIX_PER = _N_PIX // _N_WORKERS
_ICHUNK = 4096
_N_ICHUNKS = _N_SUB // _ICHUNK
_GCHUNK = 1024
_N_GCHUNKS = _PIX_PER // _GCHUNK


def _sc_scatter(idx, img):
    mesh = plsc.VectorSubcoreMesh(core_axis_name="c", subcore_axis_name="s")

    @functools.partial(
        pl.kernel,
        mesh=mesh,
        out_type=jax.ShapeDtypeStruct((16, _N_PIX), jnp.float32),
        scratch_types=[
            pltpu.VMEM((_PIX_PER,), jnp.int32),
            pltpu.VMEM((_ICHUNK,), jnp.int32),
            pltpu.VMEM((_ICHUNK,), jnp.int32),
            pltpu.VMEM((16 * _GCHUNK,), jnp.float32),
            pltpu.VMEM((16 * _GCHUNK,), jnp.float32),
            pltpu.SemaphoreType.DMA,
            pltpu.SemaphoreType.DMA,
            pltpu.SemaphoreType.DMA,
            pltpu.SemaphoreType.DMA,
            pltpu.SemaphoreType.DMA,
            pltpu.SemaphoreType.DMA,
        ],
        compiler_params=pltpu.CompilerParams(
            needs_layout_passes=False, use_tc_tiling_on_sc=False),
    )
    def body(idx_hbm, img_hbm, out_hbm, claim_v, ibuf0, ibuf1, gbuf0, gbuf1,
             isem0, isem1, gsem0, gsem1, osem0, osem1):
        sc = lax.axis_index("c")
        tid = lax.axis_index("s")
        wid = tid * 2 + sc
        base = wid * _PIX_PER
        lanes = lax.iota(jnp.int32, 16)
        neg1 = jnp.full((16,), -1, jnp.int32)

        def init_body(v, _):
            claim_v[pl.ds(v * 16, 16)] = neg1
            return _
        lax.fori_loop(0, _PIX_PER // 16, init_body, 0)

        ibufs = [ibuf0, ibuf1]
        isems = [isem0, isem1]
        pltpu.async_copy(idx_hbm.at[pl.ds(0, _ICHUNK)], ibuf0, isem0)
        for c in range(_N_ICHUNKS):
            par = c % 2
            if c + 1 < _N_ICHUNKS:
                nxt = (c + 1) % 2
                pltpu.async_copy(
                    idx_hbm.at[pl.ds((c + 1) * _ICHUNK, _ICHUNK)],
                    ibufs[nxt], isems[nxt])
            pltpu.make_async_copy(
                idx_hbm.at[pl.ds(c * _ICHUNK, _ICHUNK)],
                ibufs[par], isems[par]).wait()
            ibuf = ibufs[par]

            def scan_body(u, _, c=c, ibuf=ibuf):
                ivs = [ibuf[pl.ds((u * 4 + k) * 16, 16)] for k in range(4)]
                for k in range(4):
                    iv = ivs[k]
                    m = lax.shift_right_logical(iv, 13) == wid
                    local = iv & (_PIX_PER - 1)
                    jv = lanes + (c * _ICHUNK + (u * 4 + k) * 16)
                    plsc.store_scatter(claim_v, [local], jv, mask=m)
                return _
            lax.fori_loop(0, _ICHUNK // 64, scan_body, 0)

        gbufs = [gbuf0, gbuf1]
        gsems = [gsem0, gsem1]
        osems = [osem0, osem1]
        fzero = jnp.zeros((16,), jnp.float32)

        def zero_gbuf(par):
            gb = gbufs[par]

            def zbody(v, _):
                gb[pl.ds(v * 16, 16)] = fzero
                return _
            lax.fori_loop(0, 16 * _GCHUNK // 16, zbody, 0)

        def fire_gathers(c2):
            par = c2 % 2
            cidx = plsc.Indices(
                claim_v.at[pl.ds(c2 * _GCHUNK, _GCHUNK)], ignored_value=-1)
            for b in range(16):
                pltpu.async_copy(
                    img_hbm.at[b].at[cidx],
                    gbufs[par].at[pl.ds(b * _GCHUNK, _GCHUNK)], gsems[par])

        def drain_gathers(c2):
            par = c2 % 2
            cidx = plsc.Indices(
                claim_v.at[pl.ds(c2 * _GCHUNK, _GCHUNK)], ignored_value=-1)
            for b in range(16):
                pltpu.make_async_copy(
                    img_hbm.at[b].at[cidx],
                    gbufs[par].at[pl.ds(b * _GCHUNK, _GCHUNK)],
                    gsems[par]).wait()

        def fire_out(c2):
            par = c2 % 2
            for b in range(16):
                pltpu.async_copy(
                    gbufs[par].at[pl.ds(b * _GCHUNK, _GCHUNK)],
                    out_hbm.at[b, pl.ds(base + c2 * _GCHUNK, _GCHUNK)],
                    osems[par])

        def drain_out(c2):
            par = c2 % 2
            for b in range(16):
                pltpu.make_async_copy(
                    gbufs[par].at[pl.ds(b * _GCHUNK, _GCHUNK)],
                    out_hbm.at[b, pl.ds(base + c2 * _GCHUNK, _GCHUNK)],
                    osems[par]).wait()

        zero_gbuf(0)
        zero_gbuf(1)
        fire_gathers(0)
        for c2 in range(_N_GCHUNKS):
            drain_gathers(c2)
            fire_out(c2)
            if c2 >= 1:
                drain_out(c2 - 1)
                zero_gbuf((c2 - 1) % 2)
            if c2 + 1 < _N_GCHUNKS:
                fire_gathers(c2 + 1)
        drain_out(_N_GCHUNKS - 1)

    return body(idx, img)


def kernel(img, subsample_idx, height, width):
    idx = (subsample_idx + (height - 512) + (width - 512)).astype(jnp.int32)
    out = _sc_scatter(idx, img)
    return out.reshape(img.shape[0], 1, 512, 512)


# speedup vs baseline: 1.2938x; 1.1050x over previous
import functools

import jax
import jax.numpy as jnp
from jax import lax
from jax.experimental import pallas as pl
from jax.experimental.pallas import tpu as pltpu
from jax.experimental.pallas import tpu_sc as plsc

_N_PIX = 512 * 512
_N_SUB = _N_PIX // 2
_N_WORKERS = 32
_PIX_PER = _N_PIX // _N_WORKERS
_N_PAD = 8192
_ROWS = _N_SUB + _N_PAD
_ICHUNK = 4096
_N_ICHUNKS = _N_SUB // _ICHUNK
_GCHUNK = 1024
_N_GCHUNKS = _PIX_PER // _GCHUNK
_GSUB = 128
_JSLAB = _N_SUB // 16
_TCH = 1024
_N_TCH = _JSLAB // _TCH
_SKP = 17


def _sc_scatter(idx, img):
    mesh = plsc.VectorSubcoreMesh(core_axis_name="c", subcore_axis_name="s")

    @functools.partial(
        pl.kernel,
        mesh=mesh,
        out_type=jax.ShapeDtypeStruct((16, _N_PIX), jnp.float32),
        scratch_types=[
            pltpu.HBM((2, _ROWS, 16), jnp.float32),
            pltpu.VMEM((_PIX_PER,), jnp.int32),
            pltpu.VMEM((_ICHUNK,), jnp.int32),
            pltpu.VMEM((_ICHUNK,), jnp.int32),
            pltpu.VMEM((_GCHUNK, 16), jnp.float32),
            pltpu.VMEM((_GCHUNK, 16), jnp.float32),
            pltpu.VMEM((16 * _SKP * (_GCHUNK // 16),), jnp.float32),
            pltpu.VMEM((_TCH, 16), jnp.float32),
            pltpu.VMEM((16 * _GCHUNK,), jnp.float32),
            pltpu.VMEM((16 * _GCHUNK,), jnp.float32),
            pltpu.SemaphoreType.DMA,
            pltpu.SemaphoreType.DMA,
            pltpu.SemaphoreType.DMA,
            pltpu.SemaphoreType.DMA,
            pltpu.SemaphoreType.DMA,
            pltpu.SemaphoreType.DMA,
            pltpu.SemaphoreType.DMA,
            pltpu.SemaphoreType.DMA,
        ],
        compiler_params=pltpu.CompilerParams(
            needs_layout_passes=False, use_tc_tiling_on_sc=False),
    )
    def body(idx_hbm, img_hbm, out_hbm, imgT_hbm, claim_v, ibuf0, ibuf1,
             gbuf0, gbuf1, skew, tbufT, tbuf0, tbuf1,
             tsem, twsem, isem0, isem1, gsem0, gsem1, osem0, osem1):
        sc = lax.axis_index("c")
        tid = lax.axis_index("s")
        wid = tid * 2 + sc
        base = wid * _PIX_PER
        lanes = lax.iota(jnp.int32, 16)
        lane_skp = lanes * _SKP
        myT = imgT_hbm.at[sc]
        slab = tid * _JSLAB
        fzero = jnp.zeros((16,), jnp.float32)

        for ch in range(_N_TCH):
            j0 = slab + ch * _TCH
            for b in range(16):
                pltpu.async_copy(img_hbm.at[b, pl.ds(j0, _TCH)],
                                 tbuf0.at[pl.ds(b * _TCH, _TCH)], tsem)
            for b in range(16):
                pltpu.make_async_copy(img_hbm.at[b, pl.ds(j0, _TCH)],
                                      tbuf0.at[pl.ds(b * _TCH, _TCH)],
                                      tsem).wait()
            if ch > 0:
                pltpu.make_async_copy(
                    tbufT, myT.at[pl.ds(j0 - _TCH, _TCH)], twsem).wait()

            def skew_copy(t, _):
                b = t & 15
                jq = lax.shift_right_logical(t, 4)
                v = tbuf0[pl.ds(b * _TCH + jq * 16, 16)]
                skew[pl.ds(jq * (16 * _SKP) + b * _SKP, 16)] = v
                return _
            lax.fori_loop(0, 16 * (_TCH // 16), skew_copy, 0)

            def trg(t, _):
                jq = lax.shift_right_logical(t, 4)
                jr = t & 15
                idxv = lane_skp + (jq * (16 * _SKP) + jr)
                row = plsc.load_gather(skew, [idxv])
                tbufT[t] = row
                return _
            lax.fori_loop(0, _TCH, trg, 0)
            pltpu.async_copy(tbufT, myT.at[pl.ds(j0, _TCH)], twsem)
        pltpu.make_async_copy(
            tbufT, myT.at[pl.ds(slab + (_N_TCH - 1) * _TCH, _TCH)],
            twsem).wait()

        def zinit(v, _):
            tbufT[v] = fzero
            return _
        lax.fori_loop(0, 512, zinit, 0)
        pltpu.async_copy(tbufT.at[pl.ds(0, 512)],
                         myT.at[pl.ds(_N_SUB + tid * 512, 512)], twsem)
        pltpu.make_async_copy(tbufT.at[pl.ds(0, 512)],
                              myT.at[pl.ds(_N_SUB + tid * 512, 512)],
                              twsem).wait()

        def init_body(v, _):
            claim_v[pl.ds(v * 16, 16)] = lanes + (_N_SUB + v * 16)
            return _
        lax.fori_loop(0, _PIX_PER // 16, init_body, 0)

        ibufs = [ibuf0, ibuf1]
        isems = [isem0, isem1]
        pltpu.async_copy(idx_hbm.at[pl.ds(0, _ICHUNK)], ibuf0, isem0)
        for c in range(_N_ICHUNKS):
            par = c % 2
            if c + 1 < _N_ICHUNKS:
                nxt = (c + 1) % 2
                pltpu.async_copy(
                    idx_hbm.at[pl.ds((c + 1) * _ICHUNK, _ICHUNK)],
                    ibufs[nxt], isems[nxt])
            pltpu.make_async_copy(
                idx_hbm.at[pl.ds(c * _ICHUNK, _ICHUNK)],
                ibufs[par], isems[par]).wait()
            ibuf = ibufs[par]

            def scan_body(u, _, c=c, ibuf=ibuf):
                ivs = [ibuf[pl.ds((u * 4 + k) * 16, 16)] for k in range(4)]
                for k in range(4):
                    iv = ivs[k]
                    m = lax.shift_right_logical(iv, 13) == wid
                    local = iv & (_PIX_PER - 1)
                    jv = lanes + (c * _ICHUNK + (u * 4 + k) * 16)
                    plsc.store_scatter(claim_v, [local], jv, mask=m)
                return _
            lax.fori_loop(0, _ICHUNK // 64, scan_body, 0)

        plsc.subcore_barrier()

        gbufs = [gbuf0, gbuf1]
        gsems = [gsem0, gsem1]
        tbufs = [tbuf0, tbuf1]
        osems = [osem0, osem1]

        def fire_gathers(c2):
            par = c2 % 2
            for g in range(_GCHUNK // _GSUB):
                pltpu.async_copy(
                    myT.at[claim_v.at[pl.ds(c2 * _GCHUNK + g * _GSUB,
                                            _GSUB)]],
                    gbufs[par].at[pl.ds(g * _GSUB, _GSUB)], gsems[par])

        def drain_gathers(c2):
            par = c2 % 2
            for g in range(_GCHUNK // _GSUB):
                pltpu.make_async_copy(
                    myT.at[claim_v.at[pl.ds(c2 * _GCHUNK + g * _GSUB,
                                            _GSUB)]],
                    gbufs[par].at[pl.ds(g * _GSUB, _GSUB)], gsems[par]).wait()

        def fire_out(c2):
            par = c2 % 2
            for b in range(16):
                pltpu.async_copy(
                    tbufs[par].at[pl.ds(b * _GCHUNK, _GCHUNK)],
                    out_hbm.at[b, pl.ds(base + c2 * _GCHUNK, _GCHUNK)],
                    osems[par])

        def drain_out(c2):
            par = c2 % 2
            for b in range(16):
                pltpu.make_async_copy(
                    tbufs[par].at[pl.ds(b * _GCHUNK, _GCHUNK)],
                    out_hbm.at[b, pl.ds(base + c2 * _GCHUNK, _GCHUNK)],
                    osems[par]).wait()

        def transpose_chunk(c2):
            par = c2 % 2
            gbf = gbufs[par]
            tbf = tbufs[par]

            def skc(t, _):
                pq = lax.shift_right_logical(t, 4)
                pr = t & 15
                v = gbf[t]
                skew[pl.ds(pq * (16 * _SKP) + pr * _SKP, 16)] = v
                return _
            lax.fori_loop(0, _GCHUNK, skc, 0)

            def trb(t, _):
                b = t & 15
                pq = lax.shift_right_logical(t, 4)
                idxv = lane_skp + (pq * (16 * _SKP) + b)
                row = plsc.load_gather(skew, [idxv])
                tbf[pl.ds(b * _GCHUNK + pq * 16, 16)] = row
                return _
            lax.fori_loop(0, 16 * (_GCHUNK // 16), trb, 0)

        fire_gathers(0)
        for c2 in range(_N_GCHUNKS):
            if c2 + 1 < _N_GCHUNKS:
                fire_gathers(c2 + 1)
            drain_gathers(c2)
            if c2 >= 2:
                drain_out(c2 - 2)
            transpose_chunk(c2)
            fire_out(c2)
        drain_out(_N_GCHUNKS - 2)
        drain_out(_N_GCHUNKS - 1)

    return body(idx, img)


def kernel(img, subsample_idx, height, width):
    idx = (subsample_idx + (height - 512) + (width - 512)).astype(jnp.int32)
    out = _sc_scatter(idx, img)
    return out.reshape(img.shape[0], 1, 512, 512)


# speedup vs baseline: 2.1137x; 1.6338x over previous
import functools

import jax
import jax.numpy as jnp
from jax import lax
from jax.experimental import pallas as pl
from jax.experimental.pallas import tpu as pltpu
from jax.experimental.pallas import tpu_sc as plsc

_N_PIX = 512 * 512
_N_SUB = _N_PIX // 2
_N_WORKERS = 32
_PIX_PER = _N_PIX // _N_WORKERS
_N_PAD = 8192
_ROWS = _N_SUB + _N_PAD
_ICHUNK = 4096
_N_ICHUNKS = _N_SUB // _ICHUNK
_GCHUNK = 1024
_N_GCHUNKS = _PIX_PER // _GCHUNK
_GSUB = 128
_JSLAB = _N_SUB // 16
_TCH = 1024
_N_TCH = _JSLAB // _TCH
_SKP = 17


def _sc_scatter(idx, img):
    mesh = plsc.VectorSubcoreMesh(core_axis_name="c", subcore_axis_name="s")

    @functools.partial(
        pl.kernel,
        mesh=mesh,
        out_type=jax.ShapeDtypeStruct((16, _N_PIX), jnp.float32),
        scratch_types=[
            pltpu.HBM((2, _ROWS, 16), jnp.float32),
            pltpu.VMEM((_PIX_PER,), jnp.int32),
            pltpu.VMEM((_ICHUNK,), jnp.int32),
            pltpu.VMEM((_ICHUNK,), jnp.int32),
            pltpu.VMEM((_GCHUNK, 16), jnp.float32),
            pltpu.VMEM((_GCHUNK, 16), jnp.float32),
            pltpu.VMEM((16 * _SKP * (_GCHUNK // 16),), jnp.float32),
            pltpu.VMEM((_TCH, 16), jnp.float32),
            pltpu.VMEM((16 * _GCHUNK,), jnp.float32),
            pltpu.VMEM((16 * _GCHUNK,), jnp.float32),
            pltpu.SemaphoreType.DMA,
            pltpu.SemaphoreType.DMA,
            pltpu.SemaphoreType.DMA,
            pltpu.SemaphoreType.DMA,
            pltpu.SemaphoreType.DMA,
            pltpu.SemaphoreType.DMA,
            pltpu.SemaphoreType.DMA,
            pltpu.SemaphoreType.DMA,
        ],
        compiler_params=pltpu.CompilerParams(
            needs_layout_passes=False, use_tc_tiling_on_sc=False),
    )
    def body(idx_hbm, img_hbm, out_hbm, imgT_hbm, claim_v, ibuf0, ibuf1,
             gbuf0, gbuf1, skew, tbufT, tbuf0, tbuf1,
             tsem, twsem, isem0, isem1, gsem0, gsem1, osem0, osem1):
        sc = lax.axis_index("c")
        tid = lax.axis_index("s")
        wid = tid * 2 + sc
        base = wid * _PIX_PER
        lanes = lax.iota(jnp.int32, 16)
        lane_skp = lanes * _SKP
        myT = imgT_hbm.at[sc]
        slab = tid * _JSLAB
        fzero = jnp.zeros((16,), jnp.float32)

        tstage = [tbuf0, tbuf1]
        tssems = [tsem, gsem0]

        def t_fire(ch):
            par = ch % 2
            j0 = slab + ch * _TCH
            for b in range(16):
                pltpu.async_copy(img_hbm.at[b, pl.ds(j0, _TCH)],
                                 tstage[par].at[pl.ds(b * _TCH, _TCH)],
                                 tssems[par])

        def t_drain(ch):
            par = ch % 2
            j0 = slab + ch * _TCH
            for b in range(16):
                pltpu.make_async_copy(img_hbm.at[b, pl.ds(j0, _TCH)],
                                      tstage[par].at[pl.ds(b * _TCH, _TCH)],
                                      tssems[par]).wait()

        t_fire(0)
        for ch in range(_N_TCH):
            j0 = slab + ch * _TCH
            if ch + 1 < _N_TCH:
                t_fire(ch + 1)
            t_drain(ch)
            if ch > 0:
                pltpu.make_async_copy(
                    tbufT, myT.at[pl.ds(j0 - _TCH, _TCH)], twsem).wait()
            stg = tstage[ch % 2]

            def skew_copy(u, _, stg=stg):
                vs = []
                for k in range(4):
                    t = u * 4 + k
                    b = t & 15
                    jq = lax.shift_right_logical(t, 4)
                    vs.append((jq, b, stg[pl.ds(b * _TCH + jq * 16, 16)]))
                for jq, b, v in vs:
                    skew[pl.ds(jq * (16 * _SKP) + b * _SKP, 16)] = v
                return _
            lax.fori_loop(0, 16 * (_TCH // 16) // 4, skew_copy, 0)

            def trg(u, _):
                rows = []
                for k in range(4):
                    t = u * 4 + k
                    jq = lax.shift_right_logical(t, 4)
                    jr = t & 15
                    idxv = lane_skp + (jq * (16 * _SKP) + jr)
                    rows.append((t, plsc.load_gather(skew, [idxv])))
                for t, row in rows:
                    tbufT[t] = row
                return _
            lax.fori_loop(0, _TCH // 4, trg, 0)
            pltpu.async_copy(tbufT, myT.at[pl.ds(j0, _TCH)], twsem)
        pltpu.make_async_copy(
            tbufT, myT.at[pl.ds(slab + (_N_TCH - 1) * _TCH, _TCH)],
            twsem).wait()

        def zinit(v, _):
            tbufT[v] = fzero
            return _
        lax.fori_loop(0, 512, zinit, 0)
        pltpu.async_copy(tbufT.at[pl.ds(0, 512)],
                         myT.at[pl.ds(_N_SUB + tid * 512, 512)], twsem)
        pltpu.make_async_copy(tbufT.at[pl.ds(0, 512)],
                              myT.at[pl.ds(_N_SUB + tid * 512, 512)],
                              twsem).wait()

        def init_body(v, _):
            claim_v[pl.ds(v * 16, 16)] = lanes + (_N_SUB + v * 16)
            return _
        lax.fori_loop(0, _PIX_PER // 16, init_body, 0)

        ibufs = [ibuf0, ibuf1]
        isems = [isem0, isem1]
        pltpu.async_copy(idx_hbm.at[pl.ds(0, _ICHUNK)], ibuf0, isem0)
        for c in range(_N_ICHUNKS):
            par = c % 2
            if c + 1 < _N_ICHUNKS:
                nxt = (c + 1) % 2
                pltpu.async_copy(
                    idx_hbm.at[pl.ds((c + 1) * _ICHUNK, _ICHUNK)],
                    ibufs[nxt], isems[nxt])
            pltpu.make_async_copy(
                idx_hbm.at[pl.ds(c * _ICHUNK, _ICHUNK)],
                ibufs[par], isems[par]).wait()
            ibuf = ibufs[par]

            def scan_body(u, _, c=c, ibuf=ibuf):
                ivs = [ibuf[pl.ds((u * 4 + k) * 16, 16)] for k in range(4)]
                for k in range(4):
                    iv = ivs[k]
                    m = lax.shift_right_logical(iv, 13) == wid
                    local = iv & (_PIX_PER - 1)
                    jv = lanes + (c * _ICHUNK + (u * 4 + k) * 16)
                    plsc.store_scatter(claim_v, [local], jv, mask=m)
                return _
            lax.fori_loop(0, _ICHUNK // 64, scan_body, 0)

        plsc.subcore_barrier()

        gbufs = [gbuf0, gbuf1]
        gsems = [gsem0, gsem1]
        tbufs = [tbuf0, tbuf1]
        osems = [osem0, osem1]

        def fire_gathers(c2):
            par = c2 % 2
            for g in range(_GCHUNK // _GSUB):
                pltpu.async_copy(
                    myT.at[claim_v.at[pl.ds(c2 * _GCHUNK + g * _GSUB,
                                            _GSUB)]],
                    gbufs[par].at[pl.ds(g * _GSUB, _GSUB)], gsems[par])

        def drain_gathers(c2):
            par = c2 % 2
            for g in range(_GCHUNK // _GSUB):
                pltpu.make_async_copy(
                    myT.at[claim_v.at[pl.ds(c2 * _GCHUNK + g * _GSUB,
                                            _GSUB)]],
                    gbufs[par].at[pl.ds(g * _GSUB, _GSUB)], gsems[par]).wait()

        def fire_out(c2):
            par = c2 % 2
            for b in range(16):
                pltpu.async_copy(
                    tbufs[par].at[pl.ds(b * _GCHUNK, _GCHUNK)],
                    out_hbm.at[b, pl.ds(base + c2 * _GCHUNK, _GCHUNK)],
                    osems[par])

        def drain_out(c2):
            par = c2 % 2
            for b in range(16):
                pltpu.make_async_copy(
                    tbufs[par].at[pl.ds(b * _GCHUNK, _GCHUNK)],
                    out_hbm.at[b, pl.ds(base + c2 * _GCHUNK, _GCHUNK)],
                    osems[par]).wait()

        def transpose_chunk(c2):
            par = c2 % 2
            gbf = gbufs[par]
            tbf = tbufs[par]

            def skc(u, _, gbf=gbf):
                vs = []
                for k in range(4):
                    t = u * 4 + k
                    pq = lax.shift_right_logical(t, 4)
                    pr = t & 15
                    vs.append((pq, pr, gbf[t]))
                for pq, pr, v in vs:
                    skew[pl.ds(pq * (16 * _SKP) + pr * _SKP, 16)] = v
                return _
            lax.fori_loop(0, _GCHUNK // 4, skc, 0)

            def trb(u, _, tbf=tbf):
                rows = []
                for k in range(4):
                    t = u * 4 + k
                    b = t & 15
                    pq = lax.shift_right_logical(t, 4)
                    idxv = lane_skp + (pq * (16 * _SKP) + b)
                    rows.append((b, pq, plsc.load_gather(skew, [idxv])))
                for b, pq, row in rows:
                    tbf[pl.ds(b * _GCHUNK + pq * 16, 16)] = row
                return _
            lax.fori_loop(0, 16 * (_GCHUNK // 16) // 4, trb, 0)

        fire_gathers(0)
        for c2 in range(_N_GCHUNKS):
            if c2 + 1 < _N_GCHUNKS:
                fire_gathers(c2 + 1)
            drain_gathers(c2)
            if c2 >= 2:
                drain_out(c2 - 2)
            transpose_chunk(c2)
            fire_out(c2)
        drain_out(_N_GCHUNKS - 2)
        drain_out(_N_GCHUNKS - 1)

    return body(idx, img)


def kernel(img, subsample_idx, height, width):
    idx = (subsample_idx + (height - 512) + (width - 512)).astype(jnp.int32)
    out = _sc_scatter(idx, img)
    return out.reshape(img.shape[0], 1, 512, 512)


# speedup vs baseline: 2.1921x; 1.0371x over previous
import functools

import jax
import jax.numpy as jnp
from jax import lax
from jax.experimental import pallas as pl
from jax.experimental.pallas import tpu as pltpu
from jax.experimental.pallas import tpu_sc as plsc

_N_PIX = 512 * 512
_N_SUB = _N_PIX // 2
_N_WORKERS = 32
_PIX_PER = _N_PIX // _N_WORKERS
_N_PAD = 8192
_ROWS = _N_SUB + _N_PAD
_ICHUNK = 4096
_N_ICHUNKS = _N_SUB // _ICHUNK
_GCHUNK = 1024
_N_GCHUNKS = _PIX_PER // _GCHUNK
_GSUB = 128
_JSLAB = _N_SUB // 16
_TCH = 1024
_N_TCH = _JSLAB // _TCH
_SKP = 17


def _sc_scatter(idx, img):
    mesh = plsc.VectorSubcoreMesh(core_axis_name="c", subcore_axis_name="s")

    @functools.partial(
        pl.kernel,
        mesh=mesh,
        out_type=jax.ShapeDtypeStruct((16, _N_PIX), jnp.float32),
        scratch_types=[
            pltpu.HBM((2, _ROWS, 16), jnp.float32),
            pltpu.VMEM((_PIX_PER,), jnp.int32),
            pltpu.VMEM((_ICHUNK,), jnp.int32),
            pltpu.VMEM((_ICHUNK,), jnp.int32),
            pltpu.VMEM((_GCHUNK, 16), jnp.float32),
            pltpu.VMEM((_GCHUNK, 16), jnp.float32),
            pltpu.VMEM((16 * _SKP * (_GCHUNK // 16),), jnp.float32),
            pltpu.VMEM((_TCH, 16), jnp.float32),
            pltpu.VMEM((16 * _GCHUNK,), jnp.float32),
            pltpu.VMEM((16 * _GCHUNK,), jnp.float32),
            pltpu.SemaphoreType.DMA,
            pltpu.SemaphoreType.DMA,
            pltpu.SemaphoreType.DMA,
            pltpu.SemaphoreType.DMA,
            pltpu.SemaphoreType.DMA,
            pltpu.SemaphoreType.DMA,
            pltpu.SemaphoreType.DMA,
            pltpu.SemaphoreType.DMA,
        ],
        compiler_params=pltpu.CompilerParams(
            needs_layout_passes=False, use_tc_tiling_on_sc=False),
    )
    def body(idx_hbm, img_hbm, out_hbm, imgT_hbm, claim_v, ibuf0, ibuf1,
             gbuf0, gbuf1, skew, tbufT, tbuf0, tbuf1,
             tsem, twsem, isem0, isem1, gsem0, gsem1, osem0, osem1):
        sc = lax.axis_index("c")
        tid = lax.axis_index("s")
        wid = tid * 2 + sc
        base = wid * _PIX_PER
        lanes = lax.iota(jnp.int32, 16)
        lane_skp = lanes * _SKP
        myT = imgT_hbm.at[sc]
        slab = tid * _JSLAB
        fzero = jnp.zeros((16,), jnp.float32)

        tstage = [tbuf0, tbuf1]
        tssems = [tsem, gsem0]

        def t_fire(ch):
            par = ch % 2
            j0 = slab + ch * _TCH
            for b in range(16):
                pltpu.async_copy(img_hbm.at[b, pl.ds(j0, _TCH)],
                                 tstage[par].at[pl.ds(b * _TCH, _TCH)],
                                 tssems[par])

        def t_drain(ch):
            par = ch % 2
            j0 = slab + ch * _TCH
            for b in range(16):
                pltpu.make_async_copy(img_hbm.at[b, pl.ds(j0, _TCH)],
                                      tstage[par].at[pl.ds(b * _TCH, _TCH)],
                                      tssems[par]).wait()

        t_fire(0)
        for ch in range(_N_TCH):
            j0 = slab + ch * _TCH
            if ch + 1 < _N_TCH:
                t_fire(ch + 1)
            t_drain(ch)
            if ch > 0:
                pltpu.make_async_copy(
                    tbufT, myT.at[pl.ds(j0 - _TCH, _TCH)], twsem).wait()
            stg = tstage[ch % 2]

            def skew_copy(u, _, stg=stg):
                vs = []
                for k in range(4):
                    t = u * 4 + k
                    b = t & 15
                    jq = lax.shift_right_logical(t, 4)
                    vs.append((jq, b, stg[pl.ds(b * _TCH + jq * 16, 16)]))
                for jq, b, v in vs:
                    skew[pl.ds(jq * (16 * _SKP) + b * _SKP, 16)] = v
                return _
            lax.fori_loop(0, 16 * (_TCH // 16) // 4, skew_copy, 0)

            def trg(u, _):
                rows = []
                for k in range(4):
                    t = u * 4 + k
                    jq = lax.shift_right_logical(t, 4)
                    jr = t & 15
                    idxv = lane_skp + (jq * (16 * _SKP) + jr)
                    rows.append((t, plsc.load_gather(skew, [idxv])))
                for t, row in rows:
                    tbufT[t] = row
                return _
            lax.fori_loop(0, _TCH // 4, trg, 0)
            pltpu.async_copy(tbufT, myT.at[pl.ds(j0, _TCH)], twsem)
        pltpu.make_async_copy(
            tbufT, myT.at[pl.ds(slab + (_N_TCH - 1) * _TCH, _TCH)],
            twsem).wait()

        def zinit(v, _):
            tbufT[v] = fzero
            return _
        lax.fori_loop(0, 512, zinit, 0)
        pltpu.async_copy(tbufT.at[pl.ds(0, 512)],
                         myT.at[pl.ds(_N_SUB + tid * 512, 512)], twsem)
        pltpu.make_async_copy(tbufT.at[pl.ds(0, 512)],
                              myT.at[pl.ds(_N_SUB + tid * 512, 512)],
                              twsem).wait()

        def init_body(v, _):
            claim_v[pl.ds(v * 16, 16)] = lanes + (_N_SUB + v * 16)
            return _
        lax.fori_loop(0, _PIX_PER // 16, init_body, 0)

        ibufs = [ibuf0, ibuf1]
        isems = [isem0, isem1]
        pltpu.async_copy(idx_hbm.at[pl.ds(0, _ICHUNK)], ibuf0, isem0)
        for c in range(_N_ICHUNKS):
            par = c % 2
            if c + 1 < _N_ICHUNKS:
                nxt = (c + 1) % 2
                pltpu.async_copy(
                    idx_hbm.at[pl.ds((c + 1) * _ICHUNK, _ICHUNK)],
                    ibufs[nxt], isems[nxt])
            pltpu.make_async_copy(
                idx_hbm.at[pl.ds(c * _ICHUNK, _ICHUNK)],
                ibufs[par], isems[par]).wait()
            ibuf = ibufs[par]

            def scan_body(u, _, c=c, ibuf=ibuf):
                ivs = [ibuf[pl.ds((u * 8 + k) * 16, 16)] for k in range(8)]
                for k in range(8):
                    iv = ivs[k]
                    m = lax.shift_right_logical(iv, 13) == wid
                    local = iv & (_PIX_PER - 1)
                    jv = lanes + (c * _ICHUNK + (u * 8 + k) * 16)
                    plsc.store_scatter(claim_v, [local], jv, mask=m)
                return _
            lax.fori_loop(0, _ICHUNK // 128, scan_body, 0)

        plsc.subcore_barrier()

        gbufs = [gbuf0, gbuf1]
        gsems = [gsem0, gsem1]
        tbufs = [tbuf0, tbuf1]
        osems = [osem0, osem1]

        def fire_gathers(c2):
            par = c2 % 2
            pltpu.async_copy(
                myT.at[claim_v.at[pl.ds(c2 * _GCHUNK, _GCHUNK)]],
                gbufs[par], gsems[par])

        def drain_gathers(c2):
            par = c2 % 2
            pltpu.make_async_copy(
                myT.at[claim_v.at[pl.ds(c2 * _GCHUNK, _GCHUNK)]],
                gbufs[par], gsems[par]).wait()

        def fire_out(c2):
            par = c2 % 2
            for b in range(16):
                pltpu.async_copy(
                    tbufs[par].at[pl.ds(b * _GCHUNK, _GCHUNK)],
                    out_hbm.at[b, pl.ds(base + c2 * _GCHUNK, _GCHUNK)],
                    osems[par])

        def drain_out(c2):
            par = c2 % 2
            for b in range(16):
                pltpu.make_async_copy(
                    tbufs[par].at[pl.ds(b * _GCHUNK, _GCHUNK)],
                    out_hbm.at[b, pl.ds(base + c2 * _GCHUNK, _GCHUNK)],
                    osems[par]).wait()

        def transpose_chunk(c2):
            par = c2 % 2
            gbf = gbufs[par]
            tbf = tbufs[par]

            def skc(u, _, gbf=gbf):
                vs = []
                for k in range(4):
                    t = u * 4 + k
                    pq = lax.shift_right_logical(t, 4)
                    pr = t & 15
                    vs.append((pq, pr, gbf[t]))
                for pq, pr, v in vs:
                    skew[pl.ds(pq * (16 * _SKP) + pr * _SKP, 16)] = v
                return _
            lax.fori_loop(0, _GCHUNK // 4, skc, 0)

            def trb(u, _, tbf=tbf):
                rows = []
                for k in range(4):
                    t = u * 4 + k
                    b = t & 15
                    pq = lax.shift_right_logical(t, 4)
                    idxv = lane_skp + (pq * (16 * _SKP) + b)
                    rows.append((b, pq, plsc.load_gather(skew, [idxv])))
                for b, pq, row in rows:
                    tbf[pl.ds(b * _GCHUNK + pq * 16, 16)] = row
                return _
            lax.fori_loop(0, 16 * (_GCHUNK // 16) // 4, trb, 0)

        fire_gathers(0)
        for c2 in range(_N_GCHUNKS):
            if c2 + 1 < _N_GCHUNKS:
                fire_gathers(c2 + 1)
            drain_gathers(c2)
            if c2 >= 2:
                drain_out(c2 - 2)
            transpose_chunk(c2)
            fire_out(c2)
        drain_out(_N_GCHUNKS - 2)
        drain_out(_N_GCHUNKS - 1)

    return body(idx, img)


def kernel(img, subsample_idx, height, width):
    idx = (subsample_idx + (height - 512) + (width - 512)).astype(jnp.int32)
    out = _sc_scatter(idx, img)
    return out.reshape(img.shape[0], 1, 512, 512)
